# Initial kernel scaffold; baseline (speedup 1.0000x reference)
import jax
import jax.numpy as jnp
from jax.experimental import pallas as pl


def kernel(x, weight):
    raise NotImplementedError("write your pallas kernel here")


# speedup vs baseline: 1.2728x; 1.2728x over previous
import functools

import jax
import jax.numpy as jnp
from jax import lax
from jax.experimental import pallas as pl
from jax.experimental.pallas import tpu as pltpu
from jax.experimental.pallas import tpu_sc as plsc

_B = 16384
_H = 50
_D = 32
_TOT = _B * _H
_NC = 2
_NS = 16
_NW = _NC * _NS
_PER_W = _TOT // _NW
_G = 128
_NG = _PER_W // _G
_K = 4
_NSTEP = _NG // _K


def _body(x_hbm, w_hbm, out_hbm, idx_v, rows_v, gsem):
    wid = lax.axis_index("s") * _NC + lax.axis_index("c")
    pltpu.sync_copy(x_hbm.at[wid], idx_v)

    def step(s, carry):
        base = s * _K
        for b in range(_K):
            pltpu.async_copy(w_hbm.at[idx_v.at[base + b]], rows_v.at[b], gsem)
        for b in range(_K):
            pltpu.make_async_copy(
                w_hbm.at[idx_v.at[base + b]], rows_v.at[b], gsem
            ).wait()
        for b in range(_K):
            pltpu.sync_copy(rows_v.at[b], out_hbm.at[wid, base + b])
        return carry

    lax.fori_loop(0, _NSTEP, step, 0)


_mesh = plsc.VectorSubcoreMesh(core_axis_name="c", subcore_axis_name="s")

_call = functools.partial(
    pl.kernel,
    mesh=_mesh,
    compiler_params=pltpu.CompilerParams(use_tc_tiling_on_sc=False),
    out_type=jax.ShapeDtypeStruct((_NW, _NG, _G, _D), jnp.float32),
    scratch_types=[
        pltpu.VMEM((_NG, _G), jnp.int32),
        pltpu.VMEM((_K, _G, _D), jnp.float32),
        pltpu.SemaphoreType.DMA,
    ],
)(_body)


def kernel(x, weight):
    xi = x.reshape(_NW, _NG, _G).astype(jnp.int32)
    out = _call(xi, weight)
    return out.reshape(_B, _H, _D)


# speedup vs baseline: 1.3058x; 1.0260x over previous
import functools

import jax
import jax.numpy as jnp
from jax import lax
from jax.experimental import pallas as pl
from jax.experimental.pallas import tpu as pltpu
from jax.experimental.pallas import tpu_sc as plsc

_B = 16384
_H = 50
_D = 32
_TOT = _B * _H
_NC = 2
_NS = 16
_NW = _NC * _NS
_PER_W = _TOT // _NW
_G = 128
_NG = _PER_W // _G
_K = 10
_NSTEP = _NG // _K
_PAIRS = _NSTEP // 2


def _body(x_hbm, w_hbm, out_hbm, idx_v, rows_v, g0, g1, w0, w1):
    gs = (g0, g1)
    ws = (w0, w1)
    wid = lax.axis_index("s") * _NC + lax.axis_index("c")
    pltpu.sync_copy(x_hbm.at[wid], idx_v)

    def fire_gathers(batch, c):
        for b in range(_K):
            pltpu.async_copy(
                w_hbm.at[idx_v.at[batch * _K + b]], rows_v.at[c, b], gs[c]
            )

    def wait_write(o):
        pltpu.make_async_copy(rows_v.at[o], out_hbm.at[wid, pl.ds(0, _K)], ws[o]).wait()

    fire_gathers(0, 0)

    def pair(p, carry):
        for i in range(2):
            s = 2 * p + i
            c = i
            o = 1 - i
            for b in range(_K):
                pltpu.make_async_copy(
                    w_hbm.at[idx_v.at[b]], rows_v.at[c, b], gs[c]
                ).wait()
            pltpu.async_copy(rows_v.at[c], out_hbm.at[wid, pl.ds(s * _K, _K)], ws[c])
            if i == 0:
                @pl.when(p >= 1)
                def _():
                    wait_write(o)
                fire_gathers(s + 1, o)
            else:
                @pl.when(p < _PAIRS - 1)
                def _():
                    wait_write(o)
                    fire_gathers(s + 1, o)
        return carry

    lax.fori_loop(0, _PAIRS, pair, 0)
    wait_write(0)
    wait_write(1)


_mesh = plsc.VectorSubcoreMesh(core_axis_name="c", subcore_axis_name="s")

_call = functools.partial(
    pl.kernel,
    mesh=_mesh,
    compiler_params=pltpu.CompilerParams(use_tc_tiling_on_sc=False),
    out_type=jax.ShapeDtypeStruct((_NW, _NG, _G, _D), jnp.float32),
    scratch_types=[
        pltpu.VMEM((_NG, _G), jnp.int32),
        pltpu.VMEM((2, _K, _G, _D), jnp.float32),
        pltpu.SemaphoreType.DMA,
        pltpu.SemaphoreType.DMA,
        pltpu.SemaphoreType.DMA,
        pltpu.SemaphoreType.DMA,
    ],
)(_body)


def kernel(x, weight):
    xi = x.reshape(_NW, _NG, _G).astype(jnp.int32)
    out = _call(xi, weight)
    return out.reshape(_B, _H, _D)
